# baseline (speedup 1.0000x reference)
import numpy as np
import jax
import jax.numpy as jnp
from jax import lax
from jax.experimental import pallas as pl
from jax.experimental.pallas import tpu as pltpu
from jax.experimental.pallas import tpu_sc as plsc

_MISSING = 0.0
_THRESH = 0.4
_B, _S, _F = 4, 4096, 1024
_ROWS = _B * _S


def _rotl32(x: np.ndarray, d: int) -> np.ndarray:
    return ((x << np.uint32(d)) | (x >> np.uint32(32 - d))).astype(np.uint32)


def _threefry2x32_np(k0: int, k1: int, x0: np.ndarray, x1: np.ndarray):
    ks = [np.uint32(k0), np.uint32(k1),
          np.uint32(k0) ^ np.uint32(k1) ^ np.uint32(0x1BD11BDA)]
    rot = [[13, 15, 26, 6], [17, 29, 16, 24]]
    x0 = (x0 + ks[0]).astype(np.uint32)
    x1 = (x1 + ks[1]).astype(np.uint32)
    for i in range(5):
        for r in rot[i % 2]:
            x0 = (x0 + x1).astype(np.uint32)
            x1 = _rotl32(x1, r)
            x1 = (x1 ^ x0).astype(np.uint32)
        x0 = (x0 + ks[(i + 1) % 3]).astype(np.uint32)
        x1 = (x1 + ks[(i + 2) % 3] + np.uint32(i + 1)).astype(np.uint32)
    return x0, x1


def _compute_mask_np() -> np.ndarray:
    n = _B * _S
    b0, b1 = _threefry2x32_np(
        0, 42, np.zeros(n, dtype=np.uint32), np.arange(n, dtype=np.uint32))
    bits = b0 ^ b1
    u = ((bits >> np.uint32(9)) | np.uint32(0x3F800000)).view(np.float32)
    u = np.maximum(np.float32(0), u - np.float32(1.0))
    return (u >= _THRESH).reshape(_B, _S)


_MASK_NP = _compute_mask_np()


_BLK = 2048


def _tc_body(x_ref, m_ref, o_ref):
    o_ref[...] = jnp.where(m_ref[...] != 0, x_ref[...], _MISSING)


def _tc_call(x):
    maskf = jnp.asarray(_MASK_NP.reshape(_ROWS, 1).astype(np.float32))
    return pl.pallas_call(
        _tc_body,
        grid=(_ROWS // _BLK,),
        in_specs=[
            pl.BlockSpec((_BLK, _F), lambda i: (i, 0)),
            pl.BlockSpec((_BLK, 1), lambda i: (i, 0)),
        ],
        out_specs=pl.BlockSpec((_BLK, _F), lambda i: (i, 0)),
        out_shape=jax.ShapeDtypeStruct((_ROWS, _F), jnp.float32),
        compiler_params=pltpu.CompilerParams(vmem_limit_bytes=120 * 1024 * 1024),
    )(x, maskf)


_NC, _NS = 2, 16
_NW = _NC * _NS
_CK = 48
_CZ = 16


def _partition(idx: np.ndarray, chunk: int) -> np.ndarray:
    per = _NW * chunk
    n = len(idx)
    nchunks = max(1, -(-n // per))
    pad = nchunks * per - n
    idx = np.concatenate([idx, np.full(pad, idx[-1], dtype=idx.dtype)])
    return np.ascontiguousarray(idx.reshape(_NW, nchunks, chunk))


_MASK_FLAT = _MASK_NP.reshape(-1)
_KIDX_NP = _partition(np.nonzero(_MASK_FLAT)[0].astype(np.int32), _CK)
_ZIDX_NP = _partition(np.nonzero(~_MASK_FLAT)[0].astype(np.int32), _CZ)
_NCK = _KIDX_NP.shape[1]
_NCZ = _ZIDX_NP.shape[1]


def _sc_body(arr_ref, kidx_ref, zidx_ref, zeros_ref, out_ref,
             kidx_v, zidx_v, zbuf, buf0, buf1,
             gsem0, gsem1, ssem0, ssem1, zsem):
    wid = lax.axis_index("s") * _NC + lax.axis_index("c")
    pltpu.sync_copy(kidx_ref.at[wid], kidx_v)
    pltpu.sync_copy(zidx_ref.at[wid], zidx_v)
    pltpu.sync_copy(zeros_ref, zbuf)

    zpend = [
        pltpu.async_copy(zbuf, out_ref.at[zidx_v.at[j]], zsem)
        for j in range(_NCZ)
    ]

    bufs = (buf0, buf1)
    gsems = (gsem0, gsem1)
    ssems = (ssem0, ssem1)
    gpend = [None, None]
    spend = [None, None]
    gpend[0] = pltpu.async_copy(arr_ref.at[kidx_v.at[0]], bufs[0], gsems[0])
    for j in range(_NCK):
        b = j & 1
        gpend[b].wait()
        nxt = j + 1
        if nxt < _NCK:
            nb = nxt & 1
            if spend[nb] is not None:
                spend[nb].wait()
            gpend[nb] = pltpu.async_copy(
                arr_ref.at[kidx_v.at[nxt]], bufs[nb], gsems[nb])
        spend[b] = pltpu.async_copy(bufs[b], out_ref.at[kidx_v.at[j]], ssems[b])
    for d in spend:
        if d is not None:
            d.wait()
    for d in zpend:
        d.wait()


def _sc_call(x):
    mesh = plsc.VectorSubcoreMesh(
        core_axis_name="c", subcore_axis_name="s",
        num_cores=_NC, num_subcores=_NS)
    run = pl.kernel(
        _sc_body,
        out_type=jax.ShapeDtypeStruct((_ROWS, _F), jnp.float32),
        mesh=mesh,
        scratch_types=[
            pltpu.VMEM((_NCK, _CK), jnp.int32),
            pltpu.VMEM((_NCZ, _CZ), jnp.int32),
            pltpu.VMEM((_CZ, _F), jnp.float32),
            pltpu.VMEM((_CK, _F), jnp.float32),
            pltpu.VMEM((_CK, _F), jnp.float32),
            pltpu.SemaphoreType.DMA,
            pltpu.SemaphoreType.DMA,
            pltpu.SemaphoreType.DMA,
            pltpu.SemaphoreType.DMA,
            pltpu.SemaphoreType.DMA,
        ],
    )
    kidx = jnp.asarray(_KIDX_NP)
    zidx = jnp.asarray(_ZIDX_NP)
    zeros = jnp.zeros((_CZ, _F), jnp.float32)
    return run(x, kidx, zidx, zeros)


def kernel(arr):
    b, s, f = arr.shape
    out = _sc_call(arr.reshape(b * s, f))
    return out.reshape(b, s, f)


# speedup vs baseline: 2.8411x; 2.8411x over previous
import numpy as np
import jax
import jax.numpy as jnp
from jax import lax
from jax.experimental import pallas as pl
from jax.experimental.pallas import tpu as pltpu
from jax.experimental.pallas import tpu_sc as plsc

_MISSING = 0.0
_THRESH = 0.4
_B, _S, _F = 4, 4096, 1024
_ROWS = _B * _S


def _rotl32(x: np.ndarray, d: int) -> np.ndarray:
    return ((x << np.uint32(d)) | (x >> np.uint32(32 - d))).astype(np.uint32)


def _threefry2x32_np(k0: int, k1: int, x0: np.ndarray, x1: np.ndarray):
    ks = [np.uint32(k0), np.uint32(k1),
          np.uint32(k0) ^ np.uint32(k1) ^ np.uint32(0x1BD11BDA)]
    rot = [[13, 15, 26, 6], [17, 29, 16, 24]]
    x0 = (x0 + ks[0]).astype(np.uint32)
    x1 = (x1 + ks[1]).astype(np.uint32)
    for i in range(5):
        for r in rot[i % 2]:
            x0 = (x0 + x1).astype(np.uint32)
            x1 = _rotl32(x1, r)
            x1 = (x1 ^ x0).astype(np.uint32)
        x0 = (x0 + ks[(i + 1) % 3]).astype(np.uint32)
        x1 = (x1 + ks[(i + 2) % 3] + np.uint32(i + 1)).astype(np.uint32)
    return x0, x1


def _compute_mask_np() -> np.ndarray:
    n = _B * _S
    b0, b1 = _threefry2x32_np(
        0, 42, np.zeros(n, dtype=np.uint32), np.arange(n, dtype=np.uint32))
    bits = b0 ^ b1
    u = ((bits >> np.uint32(9)) | np.uint32(0x3F800000)).view(np.float32)
    u = np.maximum(np.float32(0), u - np.float32(1.0))
    return (u >= _THRESH).reshape(_B, _S)


_MASK_NP = _compute_mask_np()


_CH = 1024
_NI = 3
_NO = 3
_NCHUNK = _ROWS // _CH


def _ring_body(x_hbm, m_hbm, o_hbm, *scr):
    ibufs = scr[:_NI]
    obufs = scr[_NI:_NI + _NO]
    mbufs = scr[_NI + _NO:_NI + _NO + _NI]
    gsems = scr[_NI + _NO + _NI:2 * _NI + _NO + _NI]
    ssems = scr[2 * _NI + _NO + _NI:2 * _NI + 2 * _NO + _NI]
    msems = scr[2 * _NI + 2 * _NO + _NI:]

    gp = [None] * _NI
    mp = [None] * _NI
    sp = [None] * _NO
    for k in range(min(_NI, _NCHUNK)):
        gp[k] = pltpu.make_async_copy(
            x_hbm.at[pl.ds(k * _CH, _CH)], ibufs[k], gsems[k])
        gp[k].start()
        mp[k] = pltpu.make_async_copy(
            m_hbm.at[pl.ds(k * _CH, _CH)], mbufs[k], msems[k])
        mp[k].start()

    for j in range(_NCHUNK):
        bi = j % _NI
        bo = j % _NO
        if sp[bo] is not None:
            sp[bo].wait()
        gp[bi].wait()
        mp[bi].wait()
        obufs[bo][...] = jnp.where(mbufs[bi][...] != 0, ibufs[bi][...], _MISSING)
        nxt = j + _NI
        if nxt < _NCHUNK:
            gp[bi] = pltpu.make_async_copy(
                x_hbm.at[pl.ds(nxt * _CH, _CH)], ibufs[bi], gsems[bi])
            gp[bi].start()
            mp[bi] = pltpu.make_async_copy(
                m_hbm.at[pl.ds(nxt * _CH, _CH)], mbufs[bi], msems[bi])
            mp[bi].start()
        sp[bo] = pltpu.make_async_copy(
            obufs[bo], o_hbm.at[pl.ds(j * _CH, _CH)], ssems[bo])
        sp[bo].start()
    for d in sp:
        if d is not None:
            d.wait()


def _tc_call(x):
    maskf = jnp.asarray(_MASK_NP.reshape(_ROWS, 1).astype(np.float32))
    return pl.pallas_call(
        _ring_body,
        in_specs=[
            pl.BlockSpec(memory_space=pl.ANY),
            pl.BlockSpec(memory_space=pl.ANY),
        ],
        out_specs=pl.BlockSpec(memory_space=pl.ANY),
        out_shape=jax.ShapeDtypeStruct((_ROWS, _F), jnp.float32),
        scratch_shapes=(
            [pltpu.VMEM((_CH, _F), jnp.float32) for _ in range(_NI)]
            + [pltpu.VMEM((_CH, _F), jnp.float32) for _ in range(_NO)]
            + [pltpu.VMEM((_CH, 1), jnp.float32) for _ in range(_NI)]
            + [pltpu.SemaphoreType.DMA for _ in range(2 * _NI + _NO)]
        ),
        compiler_params=pltpu.CompilerParams(vmem_limit_bytes=100 * 1024 * 1024),
    )(x, maskf)


_NC, _NS = 2, 16
_NW = _NC * _NS
_CK = 48
_CZ = 16


def _partition(idx: np.ndarray, chunk: int) -> np.ndarray:
    per = _NW * chunk
    n = len(idx)
    nchunks = max(1, -(-n // per))
    pad = nchunks * per - n
    idx = np.concatenate([idx, np.full(pad, idx[-1], dtype=idx.dtype)])
    return np.ascontiguousarray(idx.reshape(_NW, nchunks, chunk))


_MASK_FLAT = _MASK_NP.reshape(-1)
_KIDX_NP = _partition(np.nonzero(_MASK_FLAT)[0].astype(np.int32), _CK)
_ZIDX_NP = _partition(np.nonzero(~_MASK_FLAT)[0].astype(np.int32), _CZ)
_NCK = _KIDX_NP.shape[1]
_NCZ = _ZIDX_NP.shape[1]


def _sc_body(arr_ref, kidx_ref, zidx_ref, zeros_ref, out_ref,
             kidx_v, zidx_v, zbuf, buf0, buf1,
             gsem0, gsem1, ssem0, ssem1, zsem):
    wid = lax.axis_index("s") * _NC + lax.axis_index("c")
    pltpu.sync_copy(kidx_ref.at[wid], kidx_v)
    pltpu.sync_copy(zidx_ref.at[wid], zidx_v)
    pltpu.sync_copy(zeros_ref, zbuf)

    zpend = [
        pltpu.async_copy(zbuf, out_ref.at[zidx_v.at[j]], zsem)
        for j in range(_NCZ)
    ]

    bufs = (buf0, buf1)
    gsems = (gsem0, gsem1)
    ssems = (ssem0, ssem1)
    gpend = [None, None]
    spend = [None, None]
    gpend[0] = pltpu.async_copy(arr_ref.at[kidx_v.at[0]], bufs[0], gsems[0])
    for j in range(_NCK):
        b = j & 1
        gpend[b].wait()
        nxt = j + 1
        if nxt < _NCK:
            nb = nxt & 1
            if spend[nb] is not None:
                spend[nb].wait()
            gpend[nb] = pltpu.async_copy(
                arr_ref.at[kidx_v.at[nxt]], bufs[nb], gsems[nb])
        spend[b] = pltpu.async_copy(bufs[b], out_ref.at[kidx_v.at[j]], ssems[b])
    for d in spend:
        if d is not None:
            d.wait()
    for d in zpend:
        d.wait()


def _sc_call(x):
    mesh = plsc.VectorSubcoreMesh(
        core_axis_name="c", subcore_axis_name="s",
        num_cores=_NC, num_subcores=_NS)
    run = pl.kernel(
        _sc_body,
        out_type=jax.ShapeDtypeStruct((_ROWS, _F), jnp.float32),
        mesh=mesh,
        scratch_types=[
            pltpu.VMEM((_NCK, _CK), jnp.int32),
            pltpu.VMEM((_NCZ, _CZ), jnp.int32),
            pltpu.VMEM((_CZ, _F), jnp.float32),
            pltpu.VMEM((_CK, _F), jnp.float32),
            pltpu.VMEM((_CK, _F), jnp.float32),
            pltpu.SemaphoreType.DMA,
            pltpu.SemaphoreType.DMA,
            pltpu.SemaphoreType.DMA,
            pltpu.SemaphoreType.DMA,
            pltpu.SemaphoreType.DMA,
        ],
    )
    kidx = jnp.asarray(_KIDX_NP)
    zidx = jnp.asarray(_ZIDX_NP)
    zeros = jnp.zeros((_CZ, _F), jnp.float32)
    return run(x, kidx, zidx, zeros)


def kernel(arr):
    b, s, f = arr.shape
    out = _tc_call(arr.reshape(b * s, f))
    return out.reshape(b, s, f)


# speedup vs baseline: 2.8438x; 1.0010x over previous
import numpy as np
import jax
import jax.numpy as jnp
from jax import lax
from jax.experimental import pallas as pl
from jax.experimental.pallas import tpu as pltpu
from jax.experimental.pallas import tpu_sc as plsc

_MISSING = 0.0
_THRESH = 0.4
_B, _S, _F = 4, 4096, 1024
_ROWS = _B * _S


def _rotl32(x: np.ndarray, d: int) -> np.ndarray:
    return ((x << np.uint32(d)) | (x >> np.uint32(32 - d))).astype(np.uint32)


def _threefry2x32_np(k0: int, k1: int, x0: np.ndarray, x1: np.ndarray):
    ks = [np.uint32(k0), np.uint32(k1),
          np.uint32(k0) ^ np.uint32(k1) ^ np.uint32(0x1BD11BDA)]
    rot = [[13, 15, 26, 6], [17, 29, 16, 24]]
    x0 = (x0 + ks[0]).astype(np.uint32)
    x1 = (x1 + ks[1]).astype(np.uint32)
    for i in range(5):
        for r in rot[i % 2]:
            x0 = (x0 + x1).astype(np.uint32)
            x1 = _rotl32(x1, r)
            x1 = (x1 ^ x0).astype(np.uint32)
        x0 = (x0 + ks[(i + 1) % 3]).astype(np.uint32)
        x1 = (x1 + ks[(i + 2) % 3] + np.uint32(i + 1)).astype(np.uint32)
    return x0, x1


def _compute_mask_np() -> np.ndarray:
    n = _B * _S
    b0, b1 = _threefry2x32_np(
        0, 42, np.zeros(n, dtype=np.uint32), np.arange(n, dtype=np.uint32))
    bits = b0 ^ b1
    u = ((bits >> np.uint32(9)) | np.uint32(0x3F800000)).view(np.float32)
    u = np.maximum(np.float32(0), u - np.float32(1.0))
    return (u >= _THRESH).reshape(_B, _S)


_MASK_NP = _compute_mask_np()


_CH = 2048
_NI = 2
_NO = 2
_NCHUNK = _ROWS // _CH


def _ring_body(x_hbm, m_hbm, o_hbm, *scr):
    ibufs = scr[:_NI]
    obufs = scr[_NI:_NI + _NO]
    mbufs = scr[_NI + _NO:_NI + _NO + _NI]
    gsems = scr[_NI + _NO + _NI:2 * _NI + _NO + _NI]
    ssems = scr[2 * _NI + _NO + _NI:2 * _NI + 2 * _NO + _NI]
    msems = scr[2 * _NI + 2 * _NO + _NI:]

    gp = [None] * _NI
    mp = [None] * _NI
    sp = [None] * _NO
    for k in range(min(_NI, _NCHUNK)):
        gp[k] = pltpu.make_async_copy(
            x_hbm.at[pl.ds(k * _CH, _CH)], ibufs[k], gsems[k])
        gp[k].start()
        mp[k] = pltpu.make_async_copy(
            m_hbm.at[pl.ds(k * _CH, _CH)], mbufs[k], msems[k])
        mp[k].start()

    for j in range(_NCHUNK):
        bi = j % _NI
        bo = j % _NO
        if sp[bo] is not None:
            sp[bo].wait()
        gp[bi].wait()
        mp[bi].wait()
        obufs[bo][...] = jnp.where(mbufs[bi][...] != 0, ibufs[bi][...], _MISSING)
        nxt = j + _NI
        if nxt < _NCHUNK:
            gp[bi] = pltpu.make_async_copy(
                x_hbm.at[pl.ds(nxt * _CH, _CH)], ibufs[bi], gsems[bi])
            gp[bi].start()
            mp[bi] = pltpu.make_async_copy(
                m_hbm.at[pl.ds(nxt * _CH, _CH)], mbufs[bi], msems[bi])
            mp[bi].start()
        sp[bo] = pltpu.make_async_copy(
            obufs[bo], o_hbm.at[pl.ds(j * _CH, _CH)], ssems[bo])
        sp[bo].start()
    for d in sp:
        if d is not None:
            d.wait()


def _tc_call(x):
    maskf = jnp.asarray(_MASK_NP.reshape(_ROWS, 1).astype(np.float32))
    return pl.pallas_call(
        _ring_body,
        in_specs=[
            pl.BlockSpec(memory_space=pl.ANY),
            pl.BlockSpec(memory_space=pl.ANY),
        ],
        out_specs=pl.BlockSpec(memory_space=pl.ANY),
        out_shape=jax.ShapeDtypeStruct((_ROWS, _F), jnp.float32),
        scratch_shapes=(
            [pltpu.VMEM((_CH, _F), jnp.float32) for _ in range(_NI)]
            + [pltpu.VMEM((_CH, _F), jnp.float32) for _ in range(_NO)]
            + [pltpu.VMEM((_CH, 1), jnp.float32) for _ in range(_NI)]
            + [pltpu.SemaphoreType.DMA for _ in range(2 * _NI + _NO)]
        ),
        compiler_params=pltpu.CompilerParams(vmem_limit_bytes=100 * 1024 * 1024),
    )(x, maskf)


_NC, _NS = 2, 16
_NW = _NC * _NS
_CK = 48
_CZ = 16


def _partition(idx: np.ndarray, chunk: int) -> np.ndarray:
    per = _NW * chunk
    n = len(idx)
    nchunks = max(1, -(-n // per))
    pad = nchunks * per - n
    idx = np.concatenate([idx, np.full(pad, idx[-1], dtype=idx.dtype)])
    return np.ascontiguousarray(idx.reshape(_NW, nchunks, chunk))


_MASK_FLAT = _MASK_NP.reshape(-1)
_KIDX_NP = _partition(np.nonzero(_MASK_FLAT)[0].astype(np.int32), _CK)
_ZIDX_NP = _partition(np.nonzero(~_MASK_FLAT)[0].astype(np.int32), _CZ)
_NCK = _KIDX_NP.shape[1]
_NCZ = _ZIDX_NP.shape[1]


def _sc_body(arr_ref, kidx_ref, zidx_ref, zeros_ref, out_ref,
             kidx_v, zidx_v, zbuf, buf0, buf1,
             gsem0, gsem1, ssem0, ssem1, zsem):
    wid = lax.axis_index("s") * _NC + lax.axis_index("c")
    pltpu.sync_copy(kidx_ref.at[wid], kidx_v)
    pltpu.sync_copy(zidx_ref.at[wid], zidx_v)
    pltpu.sync_copy(zeros_ref, zbuf)

    zpend = [
        pltpu.async_copy(zbuf, out_ref.at[zidx_v.at[j]], zsem)
        for j in range(_NCZ)
    ]

    bufs = (buf0, buf1)
    gsems = (gsem0, gsem1)
    ssems = (ssem0, ssem1)
    gpend = [None, None]
    spend = [None, None]
    gpend[0] = pltpu.async_copy(arr_ref.at[kidx_v.at[0]], bufs[0], gsems[0])
    for j in range(_NCK):
        b = j & 1
        gpend[b].wait()
        nxt = j + 1
        if nxt < _NCK:
            nb = nxt & 1
            if spend[nb] is not None:
                spend[nb].wait()
            gpend[nb] = pltpu.async_copy(
                arr_ref.at[kidx_v.at[nxt]], bufs[nb], gsems[nb])
        spend[b] = pltpu.async_copy(bufs[b], out_ref.at[kidx_v.at[j]], ssems[b])
    for d in spend:
        if d is not None:
            d.wait()
    for d in zpend:
        d.wait()


def _sc_call(x):
    mesh = plsc.VectorSubcoreMesh(
        core_axis_name="c", subcore_axis_name="s",
        num_cores=_NC, num_subcores=_NS)
    run = pl.kernel(
        _sc_body,
        out_type=jax.ShapeDtypeStruct((_ROWS, _F), jnp.float32),
        mesh=mesh,
        scratch_types=[
            pltpu.VMEM((_NCK, _CK), jnp.int32),
            pltpu.VMEM((_NCZ, _CZ), jnp.int32),
            pltpu.VMEM((_CZ, _F), jnp.float32),
            pltpu.VMEM((_CK, _F), jnp.float32),
            pltpu.VMEM((_CK, _F), jnp.float32),
            pltpu.SemaphoreType.DMA,
            pltpu.SemaphoreType.DMA,
            pltpu.SemaphoreType.DMA,
            pltpu.SemaphoreType.DMA,
            pltpu.SemaphoreType.DMA,
        ],
    )
    kidx = jnp.asarray(_KIDX_NP)
    zidx = jnp.asarray(_ZIDX_NP)
    zeros = jnp.zeros((_CZ, _F), jnp.float32)
    return run(x, kidx, zidx, zeros)


def kernel(arr):
    b, s, f = arr.shape
    out = _tc_call(arr.reshape(b * s, f))
    return out.reshape(b, s, f)


# speedup vs baseline: 2.8488x; 1.0018x over previous
import numpy as np
import jax
import jax.numpy as jnp
from jax import lax
from jax.experimental import pallas as pl
from jax.experimental.pallas import tpu as pltpu
from jax.experimental.pallas import tpu_sc as plsc

_MISSING = 0.0
_THRESH = 0.4
_B, _S, _F = 4, 4096, 1024
_ROWS = _B * _S


def _rotl32(x: np.ndarray, d: int) -> np.ndarray:
    return ((x << np.uint32(d)) | (x >> np.uint32(32 - d))).astype(np.uint32)


def _threefry2x32_np(k0: int, k1: int, x0: np.ndarray, x1: np.ndarray):
    ks = [np.uint32(k0), np.uint32(k1),
          np.uint32(k0) ^ np.uint32(k1) ^ np.uint32(0x1BD11BDA)]
    rot = [[13, 15, 26, 6], [17, 29, 16, 24]]
    x0 = (x0 + ks[0]).astype(np.uint32)
    x1 = (x1 + ks[1]).astype(np.uint32)
    for i in range(5):
        for r in rot[i % 2]:
            x0 = (x0 + x1).astype(np.uint32)
            x1 = _rotl32(x1, r)
            x1 = (x1 ^ x0).astype(np.uint32)
        x0 = (x0 + ks[(i + 1) % 3]).astype(np.uint32)
        x1 = (x1 + ks[(i + 2) % 3] + np.uint32(i + 1)).astype(np.uint32)
    return x0, x1


def _compute_mask_np() -> np.ndarray:
    n = _B * _S
    b0, b1 = _threefry2x32_np(
        0, 42, np.zeros(n, dtype=np.uint32), np.arange(n, dtype=np.uint32))
    bits = b0 ^ b1
    u = ((bits >> np.uint32(9)) | np.uint32(0x3F800000)).view(np.float32)
    u = np.maximum(np.float32(0), u - np.float32(1.0))
    return (u >= _THRESH).reshape(_B, _S)


_MASK_NP = _compute_mask_np()


_CH = 512
_NI = 6
_NO = 6
_NCHUNK = _ROWS // _CH


def _ring_body(x_hbm, m_hbm, o_hbm, *scr):
    ibufs = scr[:_NI]
    obufs = scr[_NI:_NI + _NO]
    mbufs = scr[_NI + _NO:_NI + _NO + _NI]
    gsems = scr[_NI + _NO + _NI:2 * _NI + _NO + _NI]
    ssems = scr[2 * _NI + _NO + _NI:2 * _NI + 2 * _NO + _NI]
    msems = scr[2 * _NI + 2 * _NO + _NI:]

    gp = [None] * _NI
    mp = [None] * _NI
    sp = [None] * _NO
    for k in range(min(_NI, _NCHUNK)):
        gp[k] = pltpu.make_async_copy(
            x_hbm.at[pl.ds(k * _CH, _CH)], ibufs[k], gsems[k])
        gp[k].start()
        mp[k] = pltpu.make_async_copy(
            m_hbm.at[pl.ds(k * _CH, _CH)], mbufs[k], msems[k])
        mp[k].start()

    for j in range(_NCHUNK):
        bi = j % _NI
        bo = j % _NO
        if sp[bo] is not None:
            sp[bo].wait()
        gp[bi].wait()
        mp[bi].wait()
        obufs[bo][...] = jnp.where(mbufs[bi][...] != 0, ibufs[bi][...], _MISSING)
        nxt = j + _NI
        if nxt < _NCHUNK:
            gp[bi] = pltpu.make_async_copy(
                x_hbm.at[pl.ds(nxt * _CH, _CH)], ibufs[bi], gsems[bi])
            gp[bi].start()
            mp[bi] = pltpu.make_async_copy(
                m_hbm.at[pl.ds(nxt * _CH, _CH)], mbufs[bi], msems[bi])
            mp[bi].start()
        sp[bo] = pltpu.make_async_copy(
            obufs[bo], o_hbm.at[pl.ds(j * _CH, _CH)], ssems[bo])
        sp[bo].start()
    for d in sp:
        if d is not None:
            d.wait()


def _tc_call(x):
    maskf = jnp.asarray(_MASK_NP.reshape(_ROWS, 1).astype(np.float32))
    return pl.pallas_call(
        _ring_body,
        in_specs=[
            pl.BlockSpec(memory_space=pl.ANY),
            pl.BlockSpec(memory_space=pl.ANY),
        ],
        out_specs=pl.BlockSpec(memory_space=pl.ANY),
        out_shape=jax.ShapeDtypeStruct((_ROWS, _F), jnp.float32),
        scratch_shapes=(
            [pltpu.VMEM((_CH, _F), jnp.float32) for _ in range(_NI)]
            + [pltpu.VMEM((_CH, _F), jnp.float32) for _ in range(_NO)]
            + [pltpu.VMEM((_CH, 1), jnp.float32) for _ in range(_NI)]
            + [pltpu.SemaphoreType.DMA for _ in range(2 * _NI + _NO)]
        ),
        compiler_params=pltpu.CompilerParams(vmem_limit_bytes=100 * 1024 * 1024),
    )(x, maskf)


_NC, _NS = 2, 16
_NW = _NC * _NS
_CK = 48
_CZ = 16


def _partition(idx: np.ndarray, chunk: int) -> np.ndarray:
    per = _NW * chunk
    n = len(idx)
    nchunks = max(1, -(-n // per))
    pad = nchunks * per - n
    idx = np.concatenate([idx, np.full(pad, idx[-1], dtype=idx.dtype)])
    return np.ascontiguousarray(idx.reshape(_NW, nchunks, chunk))


_MASK_FLAT = _MASK_NP.reshape(-1)
_KIDX_NP = _partition(np.nonzero(_MASK_FLAT)[0].astype(np.int32), _CK)
_ZIDX_NP = _partition(np.nonzero(~_MASK_FLAT)[0].astype(np.int32), _CZ)
_NCK = _KIDX_NP.shape[1]
_NCZ = _ZIDX_NP.shape[1]


def _sc_body(arr_ref, kidx_ref, zidx_ref, zeros_ref, out_ref,
             kidx_v, zidx_v, zbuf, buf0, buf1,
             gsem0, gsem1, ssem0, ssem1, zsem):
    wid = lax.axis_index("s") * _NC + lax.axis_index("c")
    pltpu.sync_copy(kidx_ref.at[wid], kidx_v)
    pltpu.sync_copy(zidx_ref.at[wid], zidx_v)
    pltpu.sync_copy(zeros_ref, zbuf)

    zpend = [
        pltpu.async_copy(zbuf, out_ref.at[zidx_v.at[j]], zsem)
        for j in range(_NCZ)
    ]

    bufs = (buf0, buf1)
    gsems = (gsem0, gsem1)
    ssems = (ssem0, ssem1)
    gpend = [None, None]
    spend = [None, None]
    gpend[0] = pltpu.async_copy(arr_ref.at[kidx_v.at[0]], bufs[0], gsems[0])
    for j in range(_NCK):
        b = j & 1
        gpend[b].wait()
        nxt = j + 1
        if nxt < _NCK:
            nb = nxt & 1
            if spend[nb] is not None:
                spend[nb].wait()
            gpend[nb] = pltpu.async_copy(
                arr_ref.at[kidx_v.at[nxt]], bufs[nb], gsems[nb])
        spend[b] = pltpu.async_copy(bufs[b], out_ref.at[kidx_v.at[j]], ssems[b])
    for d in spend:
        if d is not None:
            d.wait()
    for d in zpend:
        d.wait()


def _sc_call(x):
    mesh = plsc.VectorSubcoreMesh(
        core_axis_name="c", subcore_axis_name="s",
        num_cores=_NC, num_subcores=_NS)
    run = pl.kernel(
        _sc_body,
        out_type=jax.ShapeDtypeStruct((_ROWS, _F), jnp.float32),
        mesh=mesh,
        scratch_types=[
            pltpu.VMEM((_NCK, _CK), jnp.int32),
            pltpu.VMEM((_NCZ, _CZ), jnp.int32),
            pltpu.VMEM((_CZ, _F), jnp.float32),
            pltpu.VMEM((_CK, _F), jnp.float32),
            pltpu.VMEM((_CK, _F), jnp.float32),
            pltpu.SemaphoreType.DMA,
            pltpu.SemaphoreType.DMA,
            pltpu.SemaphoreType.DMA,
            pltpu.SemaphoreType.DMA,
            pltpu.SemaphoreType.DMA,
        ],
    )
    kidx = jnp.asarray(_KIDX_NP)
    zidx = jnp.asarray(_ZIDX_NP)
    zeros = jnp.zeros((_CZ, _F), jnp.float32)
    return run(x, kidx, zidx, zeros)


def kernel(arr):
    b, s, f = arr.shape
    out = _tc_call(arr.reshape(b * s, f))
    return out.reshape(b, s, f)


# speedup vs baseline: 2.8510x; 1.0008x over previous
import numpy as np
import jax
import jax.numpy as jnp
from jax import lax
from jax.experimental import pallas as pl
from jax.experimental.pallas import tpu as pltpu
from jax.experimental.pallas import tpu_sc as plsc

_MISSING = 0.0
_THRESH = 0.4
_B, _S, _F = 4, 4096, 1024
_ROWS = _B * _S


def _rotl32(x: np.ndarray, d: int) -> np.ndarray:
    return ((x << np.uint32(d)) | (x >> np.uint32(32 - d))).astype(np.uint32)


def _threefry2x32_np(k0: int, k1: int, x0: np.ndarray, x1: np.ndarray):
    ks = [np.uint32(k0), np.uint32(k1),
          np.uint32(k0) ^ np.uint32(k1) ^ np.uint32(0x1BD11BDA)]
    rot = [[13, 15, 26, 6], [17, 29, 16, 24]]
    x0 = (x0 + ks[0]).astype(np.uint32)
    x1 = (x1 + ks[1]).astype(np.uint32)
    for i in range(5):
        for r in rot[i % 2]:
            x0 = (x0 + x1).astype(np.uint32)
            x1 = _rotl32(x1, r)
            x1 = (x1 ^ x0).astype(np.uint32)
        x0 = (x0 + ks[(i + 1) % 3]).astype(np.uint32)
        x1 = (x1 + ks[(i + 2) % 3] + np.uint32(i + 1)).astype(np.uint32)
    return x0, x1


def _compute_mask_np() -> np.ndarray:
    n = _B * _S
    b0, b1 = _threefry2x32_np(
        0, 42, np.zeros(n, dtype=np.uint32), np.arange(n, dtype=np.uint32))
    bits = b0 ^ b1
    u = ((bits >> np.uint32(9)) | np.uint32(0x3F800000)).view(np.float32)
    u = np.maximum(np.float32(0), u - np.float32(1.0))
    return (u >= _THRESH).reshape(_B, _S)


_MASK_NP = _compute_mask_np()


_CH = 256
_NI = 8
_NO = 8
_NCHUNK = _ROWS // _CH


def _ring_body(x_hbm, m_hbm, o_hbm, *scr):
    ibufs = scr[:_NI]
    obufs = scr[_NI:_NI + _NO]
    mbufs = scr[_NI + _NO:_NI + _NO + _NI]
    gsems = scr[_NI + _NO + _NI:2 * _NI + _NO + _NI]
    ssems = scr[2 * _NI + _NO + _NI:2 * _NI + 2 * _NO + _NI]
    msems = scr[2 * _NI + 2 * _NO + _NI:]

    gp = [None] * _NI
    mp = [None] * _NI
    sp = [None] * _NO
    for k in range(min(_NI, _NCHUNK)):
        gp[k] = pltpu.make_async_copy(
            x_hbm.at[pl.ds(k * _CH, _CH)], ibufs[k], gsems[k])
        gp[k].start()
        mp[k] = pltpu.make_async_copy(
            m_hbm.at[pl.ds(k * _CH, _CH)], mbufs[k], msems[k])
        mp[k].start()

    for j in range(_NCHUNK):
        bi = j % _NI
        bo = j % _NO
        if sp[bo] is not None:
            sp[bo].wait()
        gp[bi].wait()
        mp[bi].wait()
        obufs[bo][...] = jnp.where(mbufs[bi][...] != 0, ibufs[bi][...], _MISSING)
        nxt = j + _NI
        if nxt < _NCHUNK:
            gp[bi] = pltpu.make_async_copy(
                x_hbm.at[pl.ds(nxt * _CH, _CH)], ibufs[bi], gsems[bi])
            gp[bi].start()
            mp[bi] = pltpu.make_async_copy(
                m_hbm.at[pl.ds(nxt * _CH, _CH)], mbufs[bi], msems[bi])
            mp[bi].start()
        sp[bo] = pltpu.make_async_copy(
            obufs[bo], o_hbm.at[pl.ds(j * _CH, _CH)], ssems[bo])
        sp[bo].start()
    for d in sp:
        if d is not None:
            d.wait()


def _tc_call(x):
    maskf = jnp.asarray(_MASK_NP.reshape(_ROWS, 1).astype(np.float32))
    return pl.pallas_call(
        _ring_body,
        in_specs=[
            pl.BlockSpec(memory_space=pl.ANY),
            pl.BlockSpec(memory_space=pl.ANY),
        ],
        out_specs=pl.BlockSpec(memory_space=pl.ANY),
        out_shape=jax.ShapeDtypeStruct((_ROWS, _F), jnp.float32),
        scratch_shapes=(
            [pltpu.VMEM((_CH, _F), jnp.float32) for _ in range(_NI)]
            + [pltpu.VMEM((_CH, _F), jnp.float32) for _ in range(_NO)]
            + [pltpu.VMEM((_CH, 1), jnp.float32) for _ in range(_NI)]
            + [pltpu.SemaphoreType.DMA for _ in range(2 * _NI + _NO)]
        ),
        compiler_params=pltpu.CompilerParams(vmem_limit_bytes=100 * 1024 * 1024),
    )(x, maskf)


_NC, _NS = 2, 16
_NW = _NC * _NS
_CK = 48
_CZ = 16


def _partition(idx: np.ndarray, chunk: int) -> np.ndarray:
    per = _NW * chunk
    n = len(idx)
    nchunks = max(1, -(-n // per))
    pad = nchunks * per - n
    idx = np.concatenate([idx, np.full(pad, idx[-1], dtype=idx.dtype)])
    return np.ascontiguousarray(idx.reshape(_NW, nchunks, chunk))


_MASK_FLAT = _MASK_NP.reshape(-1)
_KIDX_NP = _partition(np.nonzero(_MASK_FLAT)[0].astype(np.int32), _CK)
_ZIDX_NP = _partition(np.nonzero(~_MASK_FLAT)[0].astype(np.int32), _CZ)
_NCK = _KIDX_NP.shape[1]
_NCZ = _ZIDX_NP.shape[1]


def _sc_body(arr_ref, kidx_ref, zidx_ref, zeros_ref, out_ref,
             kidx_v, zidx_v, zbuf, buf0, buf1,
             gsem0, gsem1, ssem0, ssem1, zsem):
    wid = lax.axis_index("s") * _NC + lax.axis_index("c")
    pltpu.sync_copy(kidx_ref.at[wid], kidx_v)
    pltpu.sync_copy(zidx_ref.at[wid], zidx_v)
    pltpu.sync_copy(zeros_ref, zbuf)

    zpend = [
        pltpu.async_copy(zbuf, out_ref.at[zidx_v.at[j]], zsem)
        for j in range(_NCZ)
    ]

    bufs = (buf0, buf1)
    gsems = (gsem0, gsem1)
    ssems = (ssem0, ssem1)
    gpend = [None, None]
    spend = [None, None]
    gpend[0] = pltpu.async_copy(arr_ref.at[kidx_v.at[0]], bufs[0], gsems[0])
    for j in range(_NCK):
        b = j & 1
        gpend[b].wait()
        nxt = j + 1
        if nxt < _NCK:
            nb = nxt & 1
            if spend[nb] is not None:
                spend[nb].wait()
            gpend[nb] = pltpu.async_copy(
                arr_ref.at[kidx_v.at[nxt]], bufs[nb], gsems[nb])
        spend[b] = pltpu.async_copy(bufs[b], out_ref.at[kidx_v.at[j]], ssems[b])
    for d in spend:
        if d is not None:
            d.wait()
    for d in zpend:
        d.wait()


def _sc_call(x):
    mesh = plsc.VectorSubcoreMesh(
        core_axis_name="c", subcore_axis_name="s",
        num_cores=_NC, num_subcores=_NS)
    run = pl.kernel(
        _sc_body,
        out_type=jax.ShapeDtypeStruct((_ROWS, _F), jnp.float32),
        mesh=mesh,
        scratch_types=[
            pltpu.VMEM((_NCK, _CK), jnp.int32),
            pltpu.VMEM((_NCZ, _CZ), jnp.int32),
            pltpu.VMEM((_CZ, _F), jnp.float32),
            pltpu.VMEM((_CK, _F), jnp.float32),
            pltpu.VMEM((_CK, _F), jnp.float32),
            pltpu.SemaphoreType.DMA,
            pltpu.SemaphoreType.DMA,
            pltpu.SemaphoreType.DMA,
            pltpu.SemaphoreType.DMA,
            pltpu.SemaphoreType.DMA,
        ],
    )
    kidx = jnp.asarray(_KIDX_NP)
    zidx = jnp.asarray(_ZIDX_NP)
    zeros = jnp.zeros((_CZ, _F), jnp.float32)
    return run(x, kidx, zidx, zeros)


def kernel(arr):
    b, s, f = arr.shape
    out = _tc_call(arr.reshape(b * s, f))
    return out.reshape(b, s, f)
